# Initial kernel scaffold; baseline (speedup 1.0000x reference)
import jax
import jax.numpy as jnp
from jax.experimental import pallas as pl


def kernel(x, edge_index, batch, Wl0, bl0, Wr0, Wl1, bl1, Wr1, Wl2, bl2, Wr2, Wc1, bc1, Wc2, bc2):
    raise NotImplementedError("write your pallas kernel here")


# speedup vs baseline: 5.0792x; 5.0792x over previous
import functools

import jax
import jax.numpy as jnp
from jax import lax
from jax.experimental import pallas as pl
from jax.experimental.pallas import tpu as pltpu
from jax.experimental.pallas import tpu_sc as plsc

N_NODES = 10000
N_EDGES = 320000
DIM = 128
HALF = 64
N_GRAPHS = 64

NCORES = 2
NSUB = 16
NW = NCORES * NSUB
EPW = N_EDGES // NW
CHUNK = 80
NCHUNK = EPW // CHUNK
ROWS_PT = 624
TAIL_ROWS = N_NODES - NSUB * ROWS_PT
CNT_W = 16


def _fill_const(buf, rows, width, value):
    def row(i, _):
        for j in range(width // 16):
            buf[i, pl.ds(j * 16, 16)] = jnp.full((16,), value, jnp.float32)
        return 0
    lax.fori_loop(0, rows, row, 0)


def _copy_rows(src_ref, dst_ref, stage, n_rows, chunk):
    n_full = n_rows // chunk
    rem = n_rows - n_full * chunk
    for k in range(n_full):
        pltpu.sync_copy(src_ref.at[pl.ds(k * chunk, chunk)], stage)
        pltpu.sync_copy(stage, dst_ref.at[pl.ds(k * chunk, chunk)])
    if rem:
        pltpu.sync_copy(src_ref.at[pl.ds(n_full * chunk, rem)],
                        stage.at[pl.ds(0, rem)])
        pltpu.sync_copy(stage.at[pl.ds(0, rem)],
                        dst_ref.at[pl.ds(n_full * chunk, rem)])


def _zero_slice(zbuf, sh, base, is_last):
    n_full = ROWS_PT // CHUNK
    rem = ROWS_PT - n_full * CHUNK
    for k in range(n_full):
        pltpu.sync_copy(zbuf, sh.at[pl.ds(base + k * CHUNK, CHUNK)])
    if rem:
        pltpu.sync_copy(zbuf.at[pl.ds(0, rem)],
                        sh.at[pl.ds(base + n_full * CHUNK, rem)])

    @pl.when(is_last)
    def _():
        pltpu.sync_copy(zbuf.at[pl.ds(0, TAIL_ROWS)],
                        sh.at[pl.ds(NSUB * ROWS_PT, TAIL_ROWS)])


def _write_slice(sh, out_ref, stage, base, is_last):
    _copy_rows(sh.at[pl.ds(base, ROWS_PT)], out_ref.at[pl.ds(base, ROWS_PT)],
               stage, ROWS_PT, CHUNK)

    @pl.when(is_last)
    def _():
        pltpu.sync_copy(sh.at[pl.ds(NSUB * ROWS_PT, TAIL_ROWS)],
                        stage.at[pl.ds(0, TAIL_ROWS)])
        pltpu.sync_copy(stage.at[pl.ds(0, TAIL_ROWS)],
                        out_ref.at[pl.ds(NSUB * ROWS_PT, TAIL_ROWS)])


def _sc_agg_body(with_cnt, h0_hbm, h1_hbm, srcr_hbm, dstr_hbm, *rest):
    if with_cnt:
        (part_hbm, cntp_hbm, src_v, dst_v, rows_v, zbuf, ones_v, czbuf, sem,
         agg_sh, cnt_sh) = rest
    else:
        (part_hbm, src_v, dst_v, rows_v, zbuf, sem, agg_sh) = rest

    c = lax.axis_index("c")
    s = lax.axis_index("s")
    wid = c * NSUB + s
    base = s * ROWS_PT
    is_last = s == NSUB - 1

    _fill_const(zbuf, CHUNK, HALF, 0.0)
    if with_cnt:
        _fill_const(czbuf, CHUNK, CNT_W, 0.0)
        _fill_const(ones_v, CHUNK, CNT_W, 1.0)

    pltpu.sync_copy(srcr_hbm.at[wid], src_v)
    pltpu.sync_copy(dstr_hbm.at[wid], dst_v)

    for half in range(2):
        h_hbm = h0_hbm if half == 0 else h1_hbm
        count_now = with_cnt and half == 0

        _zero_slice(zbuf, agg_sh, base, is_last)
        if count_now:
            _zero_slice(czbuf, cnt_sh, base, is_last)
        plsc.subcore_barrier()

        if count_now:
            def step_c(j, _):
                pltpu.async_copy(h_hbm.at[src_v.at[j]], rows_v, sem).wait()
                pltpu.sync_copy(rows_v, agg_sh.at[dst_v.at[j]], add=True)
                pltpu.sync_copy(ones_v, cnt_sh.at[dst_v.at[j]], add=True)
                return 0
            lax.fori_loop(0, NCHUNK, step_c, 0)
        else:
            def step(j, _):
                pltpu.async_copy(h_hbm.at[src_v.at[j]], rows_v, sem).wait()
                pltpu.sync_copy(rows_v, agg_sh.at[dst_v.at[j]], add=True)
                return 0
            lax.fori_loop(0, NCHUNK, step, 0)

        plsc.subcore_barrier()

        _write_slice(agg_sh, part_hbm.at[c, half], rows_v, base, is_last)
        if count_now:
            _write_slice(cnt_sh, cntp_hbm.at[c], czbuf, base, is_last)
        plsc.subcore_barrier()


def _make_sc_agg(with_cnt):
    mesh = plsc.VectorSubcoreMesh(core_axis_name="c", subcore_axis_name="s")
    out_type = [
        jax.ShapeDtypeStruct((NCORES, 2, N_NODES, HALF), jnp.float32)]
    scratch = [
        pltpu.VMEM((NCHUNK, CHUNK), jnp.int32),
        pltpu.VMEM((NCHUNK, CHUNK), jnp.int32),
        pltpu.VMEM((CHUNK, HALF), jnp.float32),
        pltpu.VMEM((CHUNK, HALF), jnp.float32),
        pltpu.SemaphoreType.DMA,
        pltpu.VMEM_SHARED((N_NODES, HALF), jnp.float32),
    ]
    if with_cnt:
        out_type.append(
            jax.ShapeDtypeStruct((NCORES, N_NODES, CNT_W), jnp.float32))
        scratch = [
            pltpu.VMEM((NCHUNK, CHUNK), jnp.int32),
            pltpu.VMEM((NCHUNK, CHUNK), jnp.int32),
            pltpu.VMEM((CHUNK, HALF), jnp.float32),
            pltpu.VMEM((CHUNK, HALF), jnp.float32),
            pltpu.VMEM((CHUNK, CNT_W), jnp.float32),
            pltpu.VMEM((CHUNK, CNT_W), jnp.float32),
            pltpu.SemaphoreType.DMA,
            pltpu.VMEM_SHARED((N_NODES, HALF), jnp.float32),
            pltpu.VMEM_SHARED((N_NODES, CNT_W), jnp.float32),
        ]
    return pl.kernel(
        functools.partial(_sc_agg_body, with_cnt),
        out_type=out_type,
        mesh=mesh,
        scratch_types=scratch,
        compiler_params=pltpu.CompilerParams(use_tc_tiling_on_sc=False),
    )


def _invcnt_body(cntp_ref, out_ref):
    c = cntp_ref[0] + cntp_ref[1]
    val = c[:, 0:1]
    out_ref[...] = 1.0 / jnp.maximum(val, 1.0)


def _invcnt_tc(cntp):
    blk = 1000
    grid = N_NODES // blk
    return pl.pallas_call(
        _invcnt_body,
        grid=(grid,),
        in_specs=[pl.BlockSpec((NCORES, blk, CNT_W), lambda i: (0, i, 0))],
        out_specs=pl.BlockSpec((blk, 1), lambda i: (i, 0)),
        out_shape=jax.ShapeDtypeStruct((N_NODES, 1), jnp.float32),
    )(cntp)


def _combine_body(h_ref, part_ref, invc_ref, wl_ref, bl_ref, wr_ref, out_ref):
    h = jnp.concatenate([h_ref[0], h_ref[1]], axis=1)
    agg = jnp.concatenate([part_ref[0, 0] + part_ref[1, 0],
                           part_ref[0, 1] + part_ref[1, 1]], axis=1)
    agg = agg * invc_ref[...]
    out = (jnp.dot(agg, wl_ref[...], preferred_element_type=jnp.float32)
           + jnp.dot(h, wr_ref[...], preferred_element_type=jnp.float32)
           + bl_ref[...])
    out = jnp.maximum(out, 0.0)
    out_ref[0] = out[:, :HALF]
    out_ref[1] = out[:, HALF:]


def _combine_tc(h_st, part, invc, Wl, bl, Wr):
    blk = 1000
    grid = N_NODES // blk
    return pl.pallas_call(
        _combine_body,
        grid=(grid,),
        in_specs=[
            pl.BlockSpec((2, blk, HALF), lambda i: (0, i, 0)),
            pl.BlockSpec((NCORES, 2, blk, HALF), lambda i: (0, 0, i, 0)),
            pl.BlockSpec((blk, 1), lambda i: (i, 0)),
            pl.BlockSpec((DIM, DIM), lambda i: (0, 0)),
            pl.BlockSpec((1, DIM), lambda i: (0, 0)),
            pl.BlockSpec((DIM, DIM), lambda i: (0, 0)),
        ],
        out_specs=pl.BlockSpec((2, blk, HALF), lambda i: (0, i, 0)),
        out_shape=jax.ShapeDtypeStruct((2, N_NODES, HALF), jnp.float32),
    )(h_st, part, invc, Wl, bl.reshape(1, DIM), Wr)


def _pool_body(h_ref, batch_ref, wc1_ref, bc1_ref, wc2_ref, bc2_ref, out_ref):
    h = jnp.concatenate([h_ref[0], h_ref[1]], axis=1)
    b = batch_ref[...]
    gid = lax.broadcasted_iota(jnp.int32, (N_NODES, N_GRAPHS), 1)
    oh = (b == gid).astype(jnp.float32)
    gs = lax.dot_general(oh, h, (((0,), (0,)), ((), ())),
                         preferred_element_type=jnp.float32)
    cnt = jnp.sum(oh, axis=0)[:, None]
    g = gs / jnp.maximum(cnt, 1.0)
    z = jnp.maximum(
        jnp.dot(g, wc1_ref[...], preferred_element_type=jnp.float32)
        + bc1_ref[...], 0.0)
    out_ref[...] = (jnp.dot(z, wc2_ref[...], preferred_element_type=jnp.float32)
                    + bc2_ref[...])


def _pool_tc(h_st, batch, Wc1, bc1, Wc2, bc2):
    return pl.pallas_call(
        _pool_body,
        out_shape=jax.ShapeDtypeStruct((N_GRAPHS, 2), jnp.float32),
    )(h_st, batch.reshape(N_NODES, 1), Wc1, bc1.reshape(1, DIM), Wc2,
      bc2.reshape(1, 2))


def kernel(x, edge_index, batch, Wl0, bl0, Wr0, Wl1, bl1, Wr1, Wl2, bl2, Wr2,
           Wc1, bc1, Wc2, bc2):
    src = edge_index[0].reshape(NW, NCHUNK, CHUNK)
    dst = edge_index[1].reshape(NW, NCHUNK, CHUNK)
    x_st = jnp.stack([x[:, :HALF], x[:, HALF:]])

    agg_cnt = _make_sc_agg(True)
    agg = _make_sc_agg(False)

    part0, cntp = agg_cnt(x_st[0], x_st[1], src, dst)
    invc = _invcnt_tc(cntp)
    h1 = _combine_tc(x_st, part0, invc, Wl0, bl0, Wr0)
    part1 = agg(h1[0], h1[1], src, dst)[0]
    h2 = _combine_tc(h1, part1, invc, Wl1, bl1, Wr1)
    part2 = agg(h2[0], h2[1], src, dst)[0]
    h3 = _combine_tc(h2, part2, invc, Wl2, bl2, Wr2)
    return _pool_tc(h3, batch, Wc1, bc1, Wc2, bc2)


# speedup vs baseline: 9.3920x; 1.8491x over previous
import functools

import jax
import jax.numpy as jnp
from jax import lax
from jax.experimental import pallas as pl
from jax.experimental.pallas import tpu as pltpu
from jax.experimental.pallas import tpu_sc as plsc

N_NODES = 10000
N_EDGES = 320000
DIM = 128
HALF = 64
N_GRAPHS = 64

NCORES = 2
NSUB = 16
NW = NCORES * NSUB
EPW = N_EDGES // NW
CHUNK = 125
NCHUNK = EPW // CHUNK
WCH = 80
ROWS_PT = 624
TAIL_ROWS = N_NODES - NSUB * ROWS_PT
CNT_W = 16


def _fill_const(buf, rows, width, value):
    def row(i, _):
        for j in range(width // 16):
            buf[i, pl.ds(j * 16, 16)] = jnp.full((16,), value, jnp.float32)
        return 0
    lax.fori_loop(0, rows, row, 0)


def _zero_slice(zbuf, sh, base, is_last):
    n_full = ROWS_PT // WCH
    rem = ROWS_PT - n_full * WCH
    for k in range(n_full):
        pltpu.sync_copy(zbuf, sh.at[pl.ds(base + k * WCH, WCH)])
    if rem:
        pltpu.sync_copy(zbuf.at[pl.ds(0, rem)],
                        sh.at[pl.ds(base + n_full * WCH, rem)])

    @pl.when(is_last)
    def _():
        pltpu.sync_copy(zbuf.at[pl.ds(0, TAIL_ROWS)],
                        sh.at[pl.ds(NSUB * ROWS_PT, TAIL_ROWS)])


def _write_slice(sh, out_ref, stage, base, is_last):
    n_full = ROWS_PT // WCH
    rem = ROWS_PT - n_full * WCH
    for k in range(n_full):
        pltpu.sync_copy(sh.at[pl.ds(base + k * WCH, WCH)], stage)
        pltpu.sync_copy(stage, out_ref.at[pl.ds(base + k * WCH, WCH)])
    if rem:
        pltpu.sync_copy(sh.at[pl.ds(base + n_full * WCH, rem)],
                        stage.at[pl.ds(0, rem)])
        pltpu.sync_copy(stage.at[pl.ds(0, rem)],
                        out_ref.at[pl.ds(base + n_full * WCH, rem)])

    @pl.when(is_last)
    def _():
        pltpu.sync_copy(sh.at[pl.ds(NSUB * ROWS_PT, TAIL_ROWS)],
                        stage.at[pl.ds(0, TAIL_ROWS)])
        pltpu.sync_copy(stage.at[pl.ds(0, TAIL_ROWS)],
                        out_ref.at[pl.ds(NSUB * ROWS_PT, TAIL_ROWS)])


def _sc_agg_body(with_cnt, h0_hbm, h1_hbm, srcr_hbm, dstr_hbm, *rest):
    if with_cnt:
        (part_hbm, cntp_hbm, src_v, dst_v, rows_a, rows_b, stage, zbuf,
         ones_v, czbuf, sem_a, sem_b, agg_sh, cnt_sh) = rest
    else:
        (part_hbm, src_v, dst_v, rows_a, rows_b, stage, zbuf, sem_a, sem_b,
         agg_sh) = rest

    c = lax.axis_index("c")
    s = lax.axis_index("s")
    wid = c * NSUB + s
    base = s * ROWS_PT
    is_last = s == NSUB - 1

    _fill_const(zbuf, WCH, HALF, 0.0)
    if with_cnt:
        _fill_const(czbuf, WCH, CNT_W, 0.0)
        _fill_const(ones_v, CHUNK, CNT_W, 1.0)

    pltpu.sync_copy(srcr_hbm.at[wid], src_v)
    pltpu.sync_copy(dstr_hbm.at[wid], dst_v)

    for half in range(2):
        h_hbm = h0_hbm if half == 0 else h1_hbm
        count_now = with_cnt and half == 0

        _zero_slice(zbuf, agg_sh, base, is_last)
        if count_now:
            _zero_slice(czbuf, cnt_sh, base, is_last)
        pltpu.async_copy(h_hbm.at[src_v.at[0]], rows_a, sem_a)
        plsc.subcore_barrier()

        def pair(i, _):
            j0 = 2 * i
            pltpu.async_copy(h_hbm.at[src_v.at[j0 + 1]], rows_b, sem_b)
            pltpu.make_async_copy(h_hbm.at[src_v.at[j0]], rows_a,
                                  sem_a).wait()
            pltpu.sync_copy(rows_a, agg_sh.at[dst_v.at[j0]], add=True)
            if count_now:
                pltpu.sync_copy(ones_v, cnt_sh.at[dst_v.at[j0]], add=True)

            @pl.when(j0 + 2 < NCHUNK)
            def _():
                pltpu.async_copy(h_hbm.at[src_v.at[j0 + 2]], rows_a, sem_a)
            pltpu.make_async_copy(h_hbm.at[src_v.at[j0 + 1]], rows_b,
                                  sem_b).wait()
            pltpu.sync_copy(rows_b, agg_sh.at[dst_v.at[j0 + 1]], add=True)
            if count_now:
                pltpu.sync_copy(ones_v, cnt_sh.at[dst_v.at[j0 + 1]], add=True)
            return 0
        lax.fori_loop(0, NCHUNK // 2, pair, 0)

        plsc.subcore_barrier()

        _write_slice(agg_sh, part_hbm.at[c, half], stage, base, is_last)
        if count_now:
            _write_slice(cnt_sh, cntp_hbm.at[c], czbuf, base, is_last)
        plsc.subcore_barrier()


def _make_sc_agg(with_cnt):
    mesh = plsc.VectorSubcoreMesh(core_axis_name="c", subcore_axis_name="s")
    out_type = [
        jax.ShapeDtypeStruct((NCORES, 2, N_NODES, HALF), jnp.float32)]
    scratch = [
        pltpu.VMEM((NCHUNK, CHUNK), jnp.int32),
        pltpu.VMEM((NCHUNK, CHUNK), jnp.int32),
        pltpu.VMEM((CHUNK, HALF), jnp.float32),
        pltpu.VMEM((CHUNK, HALF), jnp.float32),
        pltpu.VMEM((WCH, HALF), jnp.float32),
        pltpu.VMEM((WCH, HALF), jnp.float32),
        pltpu.SemaphoreType.DMA,
        pltpu.SemaphoreType.DMA,
        pltpu.VMEM_SHARED((N_NODES, HALF), jnp.float32),
    ]
    if with_cnt:
        out_type.append(
            jax.ShapeDtypeStruct((NCORES, N_NODES, CNT_W), jnp.float32))
        scratch = scratch[:6] + [
            pltpu.VMEM((CHUNK, CNT_W), jnp.float32),
            pltpu.VMEM((WCH, CNT_W), jnp.float32),
        ] + scratch[6:] + [
            pltpu.VMEM_SHARED((N_NODES, CNT_W), jnp.float32),
        ]
    return pl.kernel(
        functools.partial(_sc_agg_body, with_cnt),
        out_type=out_type,
        mesh=mesh,
        scratch_types=scratch,
        compiler_params=pltpu.CompilerParams(use_tc_tiling_on_sc=False),
    )


def _combine_body(first, h0_ref, h1_ref, part_ref, aux_ref, wl_ref, bl_ref,
                  wr_ref, o0_ref, o1_ref, *maybe_invc_out):
    h = jnp.concatenate([h0_ref[...], h1_ref[...]], axis=1)
    agg = jnp.concatenate([part_ref[0, 0] + part_ref[1, 0],
                           part_ref[0, 1] + part_ref[1, 1]], axis=1)
    if first:
        cnt = (aux_ref[0] + aux_ref[1])[:, 0:1]
        invc = 1.0 / jnp.maximum(cnt, 1.0)
        maybe_invc_out[0][...] = invc
    else:
        invc = aux_ref[...]
    agg = agg * invc
    out = (jnp.dot(agg, wl_ref[...], preferred_element_type=jnp.float32)
           + jnp.dot(h, wr_ref[...], preferred_element_type=jnp.float32)
           + bl_ref[...])
    out = jnp.maximum(out, 0.0)
    o0_ref[...] = out[:, :HALF]
    o1_ref[...] = out[:, HALF:]


def _combine_tc(first, h0, h1, part, aux, Wl, bl, Wr):
    blk = 1000
    grid = N_NODES // blk
    if first:
        aux_spec = pl.BlockSpec((NCORES, blk, CNT_W), lambda i: (0, i, 0))
    else:
        aux_spec = pl.BlockSpec((blk, 1), lambda i: (i, 0))
    out_shape = [jax.ShapeDtypeStruct((N_NODES, HALF), jnp.float32),
                 jax.ShapeDtypeStruct((N_NODES, HALF), jnp.float32)]
    out_specs = [pl.BlockSpec((blk, HALF), lambda i: (i, 0)),
                 pl.BlockSpec((blk, HALF), lambda i: (i, 0))]
    if first:
        out_shape.append(jax.ShapeDtypeStruct((N_NODES, 1), jnp.float32))
        out_specs.append(pl.BlockSpec((blk, 1), lambda i: (i, 0)))
    return pl.pallas_call(
        functools.partial(_combine_body, first),
        grid=(grid,),
        in_specs=[
            pl.BlockSpec((blk, HALF), lambda i: (i, 0)),
            pl.BlockSpec((blk, HALF), lambda i: (i, 0)),
            pl.BlockSpec((NCORES, 2, blk, HALF), lambda i: (0, 0, i, 0)),
            aux_spec,
            pl.BlockSpec((DIM, DIM), lambda i: (0, 0)),
            pl.BlockSpec((1, DIM), lambda i: (0, 0)),
            pl.BlockSpec((DIM, DIM), lambda i: (0, 0)),
        ],
        out_specs=out_specs,
        out_shape=out_shape,
    )(h0, h1, part, aux, Wl, bl.reshape(1, DIM), Wr)


def _pool_body(h0_ref, h1_ref, batch_ref, wc1_ref, bc1_ref, wc2_ref, bc2_ref,
               out_ref):
    h = jnp.concatenate([h0_ref[...], h1_ref[...]], axis=1)
    b = batch_ref[...]
    gid = lax.broadcasted_iota(jnp.int32, (N_NODES, N_GRAPHS), 1)
    oh = (b == gid).astype(jnp.float32)
    gs = lax.dot_general(oh, h, (((0,), (0,)), ((), ())),
                         preferred_element_type=jnp.float32)
    cnt = jnp.sum(oh, axis=0)[:, None]
    g = gs / jnp.maximum(cnt, 1.0)
    z = jnp.maximum(
        jnp.dot(g, wc1_ref[...], preferred_element_type=jnp.float32)
        + bc1_ref[...], 0.0)
    out_ref[...] = (jnp.dot(z, wc2_ref[...], preferred_element_type=jnp.float32)
                    + bc2_ref[...])


def _pool_tc(h0, h1, batch, Wc1, bc1, Wc2, bc2):
    return pl.pallas_call(
        _pool_body,
        out_shape=jax.ShapeDtypeStruct((N_GRAPHS, 2), jnp.float32),
    )(h0, h1, batch.reshape(N_NODES, 1), Wc1, bc1.reshape(1, DIM), Wc2,
      bc2.reshape(1, 2))


def kernel(x, edge_index, batch, Wl0, bl0, Wr0, Wl1, bl1, Wr1, Wl2, bl2, Wr2,
           Wc1, bc1, Wc2, bc2):
    src = edge_index[0].reshape(NW, NCHUNK, CHUNK)
    dst = edge_index[1].reshape(NW, NCHUNK, CHUNK)
    x0 = x[:, :HALF]
    x1 = x[:, HALF:]

    agg_cnt = _make_sc_agg(True)
    agg = _make_sc_agg(False)

    part0, cntp = agg_cnt(x0, x1, src, dst)
    h10, h11, invc = _combine_tc(True, x0, x1, part0, cntp, Wl0, bl0, Wr0)
    part1 = agg(h10, h11, src, dst)[0]
    h20, h21 = _combine_tc(False, h10, h11, part1, invc, Wl1, bl1, Wr1)
    part2 = agg(h20, h21, src, dst)[0]
    h30, h31 = _combine_tc(False, h20, h21, part2, invc, Wl2, bl2, Wr2)
    return _pool_tc(h30, h31, batch, Wc1, bc1, Wc2, bc2)


# speedup vs baseline: 11.3889x; 1.2126x over previous
import functools

import jax
import jax.numpy as jnp
from jax import lax
from jax.experimental import pallas as pl
from jax.experimental.pallas import tpu as pltpu
from jax.experimental.pallas import tpu_sc as plsc

N_NODES = 10000
N_EDGES = 320000
DIM = 128
HALF = 64
N_GRAPHS = 64

NCORES = 2
NSUB = 16
NW = NCORES * NSUB
EPW = N_EDGES // NW
CHUNK = 125
NCHUNK = EPW // CHUNK
WCH = 80
ROWS_PT = 624
TAIL_ROWS = N_NODES - NSUB * ROWS_PT
CNT_W = 16


def _fill_const(buf, rows, width, value):
    def row(i, _):
        for j in range(width // 16):
            buf[i, pl.ds(j * 16, 16)] = jnp.full((16,), value, jnp.float32)
        return 0
    lax.fori_loop(0, rows, row, 0)


def _zero_slice(zbuf, sh, base, is_last):
    n_full = ROWS_PT // WCH
    rem = ROWS_PT - n_full * WCH
    for k in range(n_full):
        pltpu.sync_copy(zbuf, sh.at[pl.ds(base + k * WCH, WCH)])
    if rem:
        pltpu.sync_copy(zbuf.at[pl.ds(0, rem)],
                        sh.at[pl.ds(base + n_full * WCH, rem)])

    @pl.when(is_last)
    def _():
        pltpu.sync_copy(zbuf.at[pl.ds(0, TAIL_ROWS)],
                        sh.at[pl.ds(NSUB * ROWS_PT, TAIL_ROWS)])


def _write_slice(sh, out_ref, stage, base, is_last):
    n_full = ROWS_PT // WCH
    rem = ROWS_PT - n_full * WCH
    for k in range(n_full):
        pltpu.sync_copy(sh.at[pl.ds(base + k * WCH, WCH)], stage)
        pltpu.sync_copy(stage, out_ref.at[pl.ds(base + k * WCH, WCH)])
    if rem:
        pltpu.sync_copy(sh.at[pl.ds(base + n_full * WCH, rem)],
                        stage.at[pl.ds(0, rem)])
        pltpu.sync_copy(stage.at[pl.ds(0, rem)],
                        out_ref.at[pl.ds(base + n_full * WCH, rem)])

    @pl.when(is_last)
    def _():
        pltpu.sync_copy(sh.at[pl.ds(NSUB * ROWS_PT, TAIL_ROWS)],
                        stage.at[pl.ds(0, TAIL_ROWS)])
        pltpu.sync_copy(stage.at[pl.ds(0, TAIL_ROWS)],
                        out_ref.at[pl.ds(NSUB * ROWS_PT, TAIL_ROWS)])


NBUF = 4


def _sc_agg_body(with_cnt, h0_hbm, h1_hbm, srcr_hbm, dstr_hbm, *rest):
    if with_cnt:
        (part_hbm, cntp_hbm, src_v, dst_v, r0, r1, r2, r3, stage, zbuf,
         ones_v, czbuf, s0, s1, s2, s3, agg_sh, cnt_sh) = rest
    else:
        (part_hbm, src_v, dst_v, r0, r1, r2, r3, stage, zbuf, s0, s1, s2, s3,
         agg_sh) = rest
    rows = (r0, r1, r2, r3)
    sems = (s0, s1, s2, s3)

    c = lax.axis_index("c")
    s = lax.axis_index("s")
    wid = c * NSUB + s
    base = s * ROWS_PT
    is_last = s == NSUB - 1

    _fill_const(zbuf, WCH, HALF, 0.0)
    if with_cnt:
        _fill_const(czbuf, WCH, CNT_W, 0.0)
        _fill_const(ones_v, CHUNK, CNT_W, 1.0)

    pltpu.sync_copy(srcr_hbm.at[wid], src_v)
    pltpu.sync_copy(dstr_hbm.at[wid], dst_v)

    for half in range(2):
        h_hbm = h0_hbm if half == 0 else h1_hbm
        count_now = with_cnt and half == 0

        _zero_slice(zbuf, agg_sh, base, is_last)
        if count_now:
            _zero_slice(czbuf, cnt_sh, base, is_last)
        for b in range(NBUF):
            pltpu.async_copy(h_hbm.at[src_v.at[b]], rows[b], sems[b])
        plsc.subcore_barrier()

        def group(i, _):
            j0 = NBUF * i
            for b in range(NBUF):
                j = j0 + b
                pltpu.make_async_copy(h_hbm.at[src_v.at[j]], rows[b],
                                      sems[b]).wait()
                pltpu.sync_copy(rows[b], agg_sh.at[dst_v.at[j]], add=True)
                if count_now:
                    pltpu.sync_copy(ones_v, cnt_sh.at[dst_v.at[j]], add=True)

                @pl.when(j + NBUF < NCHUNK)
                def _():
                    pltpu.async_copy(h_hbm.at[src_v.at[j + NBUF]], rows[b],
                                     sems[b])
            return 0
        lax.fori_loop(0, NCHUNK // NBUF, group, 0)

        plsc.subcore_barrier()

        _write_slice(agg_sh, part_hbm.at[c, half], stage, base, is_last)
        if count_now:
            _write_slice(cnt_sh, cntp_hbm.at[c], czbuf, base, is_last)
        plsc.subcore_barrier()


def _make_sc_agg(with_cnt):
    mesh = plsc.VectorSubcoreMesh(core_axis_name="c", subcore_axis_name="s")
    out_type = [
        jax.ShapeDtypeStruct((NCORES, 2, N_NODES, HALF), jnp.float32)]
    scratch = [
        pltpu.VMEM((NCHUNK, CHUNK), jnp.int32),
        pltpu.VMEM((NCHUNK, CHUNK), jnp.int32),
        pltpu.VMEM((CHUNK, HALF), jnp.float32),
        pltpu.VMEM((CHUNK, HALF), jnp.float32),
        pltpu.VMEM((CHUNK, HALF), jnp.float32),
        pltpu.VMEM((CHUNK, HALF), jnp.float32),
        pltpu.VMEM((WCH, HALF), jnp.float32),
        pltpu.VMEM((WCH, HALF), jnp.float32),
        pltpu.SemaphoreType.DMA,
        pltpu.SemaphoreType.DMA,
        pltpu.SemaphoreType.DMA,
        pltpu.SemaphoreType.DMA,
        pltpu.VMEM_SHARED((N_NODES, HALF), jnp.float32),
    ]
    if with_cnt:
        out_type.append(
            jax.ShapeDtypeStruct((NCORES, N_NODES, CNT_W), jnp.float32))
        scratch = scratch[:8] + [
            pltpu.VMEM((CHUNK, CNT_W), jnp.float32),
            pltpu.VMEM((WCH, CNT_W), jnp.float32),
        ] + scratch[8:] + [
            pltpu.VMEM_SHARED((N_NODES, CNT_W), jnp.float32),
        ]
    return pl.kernel(
        functools.partial(_sc_agg_body, with_cnt),
        out_type=out_type,
        mesh=mesh,
        scratch_types=scratch,
        compiler_params=pltpu.CompilerParams(use_tc_tiling_on_sc=False),
    )


def _combine_body(first, h0_ref, h1_ref, part_ref, aux_ref, wl_ref, bl_ref,
                  wr_ref, o0_ref, o1_ref, *maybe_invc_out):
    h = jnp.concatenate([h0_ref[...], h1_ref[...]], axis=1)
    agg = jnp.concatenate([part_ref[0, 0] + part_ref[1, 0],
                           part_ref[0, 1] + part_ref[1, 1]], axis=1)
    if first:
        cnt = (aux_ref[0] + aux_ref[1])[:, 0:1]
        invc = 1.0 / jnp.maximum(cnt, 1.0)
        maybe_invc_out[0][...] = invc
    else:
        invc = aux_ref[...]
    agg = agg * invc
    out = (jnp.dot(agg, wl_ref[...], preferred_element_type=jnp.float32)
           + jnp.dot(h, wr_ref[...], preferred_element_type=jnp.float32)
           + bl_ref[...])
    out = jnp.maximum(out, 0.0)
    o0_ref[...] = out[:, :HALF]
    o1_ref[...] = out[:, HALF:]


def _combine_tc(first, h0, h1, part, aux, Wl, bl, Wr):
    blk = 1000
    grid = N_NODES // blk
    if first:
        aux_spec = pl.BlockSpec((NCORES, blk, CNT_W), lambda i: (0, i, 0))
    else:
        aux_spec = pl.BlockSpec((blk, 1), lambda i: (i, 0))
    out_shape = [jax.ShapeDtypeStruct((N_NODES, HALF), jnp.float32),
                 jax.ShapeDtypeStruct((N_NODES, HALF), jnp.float32)]
    out_specs = [pl.BlockSpec((blk, HALF), lambda i: (i, 0)),
                 pl.BlockSpec((blk, HALF), lambda i: (i, 0))]
    if first:
        out_shape.append(jax.ShapeDtypeStruct((N_NODES, 1), jnp.float32))
        out_specs.append(pl.BlockSpec((blk, 1), lambda i: (i, 0)))
    return pl.pallas_call(
        functools.partial(_combine_body, first),
        grid=(grid,),
        in_specs=[
            pl.BlockSpec((blk, HALF), lambda i: (i, 0)),
            pl.BlockSpec((blk, HALF), lambda i: (i, 0)),
            pl.BlockSpec((NCORES, 2, blk, HALF), lambda i: (0, 0, i, 0)),
            aux_spec,
            pl.BlockSpec((DIM, DIM), lambda i: (0, 0)),
            pl.BlockSpec((1, DIM), lambda i: (0, 0)),
            pl.BlockSpec((DIM, DIM), lambda i: (0, 0)),
        ],
        out_specs=out_specs,
        out_shape=out_shape,
    )(h0, h1, part, aux, Wl, bl.reshape(1, DIM), Wr)


def _last_body(nblk, h0_ref, h1_ref, part_ref, invc_ref, wl_ref, bl_ref,
               wr_ref, batch_ref, wc1_ref, bc1_ref, wc2_ref, bc2_ref, out_ref,
               gsum, cntg):
    i = pl.program_id(0)
    blk = h0_ref.shape[0]

    @pl.when(i == 0)
    def _():
        gsum[...] = jnp.zeros_like(gsum)
        cntg[...] = jnp.zeros_like(cntg)

    h = jnp.concatenate([h0_ref[...], h1_ref[...]], axis=1)
    agg = jnp.concatenate([part_ref[0, 0] + part_ref[1, 0],
                           part_ref[0, 1] + part_ref[1, 1]], axis=1)
    agg = agg * invc_ref[...]
    h3 = (jnp.dot(agg, wl_ref[...], preferred_element_type=jnp.float32)
          + jnp.dot(h, wr_ref[...], preferred_element_type=jnp.float32)
          + bl_ref[...])
    h3 = jnp.maximum(h3, 0.0)

    b = batch_ref[...]
    gid = lax.broadcasted_iota(jnp.int32, (blk, N_GRAPHS), 1)
    oh = (b == gid).astype(jnp.float32)
    gsum[...] += lax.dot_general(oh, h3, (((0,), (0,)), ((), ())),
                                 preferred_element_type=jnp.float32)
    cntg[...] += jnp.sum(oh, axis=0)[:, None]

    @pl.when(i == nblk - 1)
    def _():
        g = gsum[...] / jnp.maximum(cntg[...], 1.0)
        z = jnp.maximum(
            jnp.dot(g, wc1_ref[...], preferred_element_type=jnp.float32)
            + bc1_ref[...], 0.0)
        out_ref[...] = (
            jnp.dot(z, wc2_ref[...], preferred_element_type=jnp.float32)
            + bc2_ref[...])


def _last_tc(h0, h1, part, invc, Wl, bl, Wr, batch, Wc1, bc1, Wc2, bc2):
    blk = 1000
    grid = N_NODES // blk
    return pl.pallas_call(
        functools.partial(_last_body, grid),
        grid=(grid,),
        in_specs=[
            pl.BlockSpec((blk, HALF), lambda i: (i, 0)),
            pl.BlockSpec((blk, HALF), lambda i: (i, 0)),
            pl.BlockSpec((NCORES, 2, blk, HALF), lambda i: (0, 0, i, 0)),
            pl.BlockSpec((blk, 1), lambda i: (i, 0)),
            pl.BlockSpec((DIM, DIM), lambda i: (0, 0)),
            pl.BlockSpec((1, DIM), lambda i: (0, 0)),
            pl.BlockSpec((DIM, DIM), lambda i: (0, 0)),
            pl.BlockSpec((blk, 1), lambda i: (i, 0)),
            pl.BlockSpec((DIM, DIM), lambda i: (0, 0)),
            pl.BlockSpec((1, DIM), lambda i: (0, 0)),
            pl.BlockSpec((DIM, 2), lambda i: (0, 0)),
            pl.BlockSpec((1, 2), lambda i: (0, 0)),
        ],
        out_specs=pl.BlockSpec((N_GRAPHS, 2), lambda i: (0, 0)),
        out_shape=jax.ShapeDtypeStruct((N_GRAPHS, 2), jnp.float32),
        scratch_shapes=[pltpu.VMEM((N_GRAPHS, DIM), jnp.float32),
                        pltpu.VMEM((N_GRAPHS, 1), jnp.float32)],
    )(h0, h1, part, invc, Wl, bl.reshape(1, DIM), Wr,
      batch.reshape(N_NODES, 1), Wc1, bc1.reshape(1, DIM), Wc2,
      bc2.reshape(1, 2))


def kernel(x, edge_index, batch, Wl0, bl0, Wr0, Wl1, bl1, Wr1, Wl2, bl2, Wr2,
           Wc1, bc1, Wc2, bc2):
    src = edge_index[0].reshape(NW, NCHUNK, CHUNK)
    dst = edge_index[1].reshape(NW, NCHUNK, CHUNK)
    x0 = x[:, :HALF]
    x1 = x[:, HALF:]

    agg_cnt = _make_sc_agg(True)
    agg = _make_sc_agg(False)

    part0, cntp = agg_cnt(x0, x1, src, dst)
    h10, h11, invc = _combine_tc(True, x0, x1, part0, cntp, Wl0, bl0, Wr0)
    part1 = agg(h10, h11, src, dst)[0]
    h20, h21 = _combine_tc(False, h10, h11, part1, invc, Wl1, bl1, Wr1)
    part2 = agg(h20, h21, src, dst)[0]
    return _last_tc(h20, h21, part2, invc, Wl2, bl2, Wr2, batch,
                    Wc1, bc1, Wc2, bc2)
